# baseline (speedup 1.0000x reference)
import functools

import jax
import jax.numpy as jnp
from jax import lax
from jax.experimental import pallas as pl
from jax.experimental.pallas import tpu as pltpu
from jax.experimental.pallas import tpu_sc as plsc

N = 10000
E = 160000
F_IN = 256
POP = 1024
HID = 512

NC = 2
NS = 16
NW = NC * NS
NP = 10240
CH = 512
EP = 163840
CPW = EP // NW // CH
ROWS_PER_TILE = NP // NS
DCH = 128

_f32 = jnp.float32


def _ew_body(pop_ref, f11w_ref, f11b_ref, f22w_ref, f22b_ref, x_ref, w1_ref,
             out_ref, xw1_ref, h2_ref):
    i = pl.program_id(0)

    @pl.when(i == 0)
    def _():
        h2 = lax.dot_general(pop_ref[...], f11w_ref[...], (((1,), (1,)), ((), ())),
                             preferred_element_type=_f32)
        h2_ref[...] = h2 + f11b_ref[...]
        xw = lax.dot_general(x_ref[...], w1_ref[...], (((1,), (1,)), ((), ())),
                             preferred_element_type=_f32)
        xw1_ref[pl.ds(0, N), :] = xw
        xw1_ref[pl.ds(N, NP - N), :] = jnp.zeros((NP - N, 32), _f32)

    y = lax.dot_general(h2_ref[...], f22w_ref[...], (((1,), (1,)), ((), ())),
                        preferred_element_type=_f32)
    out_ref[...] = jax.nn.sigmoid(y + f22b_ref[...])


def _edge_weights_xw1(population, fc11_W, fc11_b, fc22_W, fc22_b, x, conv1_W):
    RB = 6400
    grid = E // RB
    return pl.pallas_call(
        _ew_body,
        grid=(grid,),
        in_specs=[
            pl.BlockSpec((1, POP), lambda i: (0, 0)),
            pl.BlockSpec((HID, POP), lambda i: (0, 0)),
            pl.BlockSpec((1, HID), lambda i: (0, 0)),
            pl.BlockSpec((RB, HID), lambda i: (i, 0)),
            pl.BlockSpec((1, RB), lambda i: (0, i)),
            pl.BlockSpec((N, F_IN), lambda i: (0, 0)),
            pl.BlockSpec((32, F_IN), lambda i: (0, 0)),
        ],
        out_specs=(pl.BlockSpec((1, RB), lambda i: (0, i)),
                   pl.BlockSpec((NP, 32), lambda i: (0, 0))),
        out_shape=(jax.ShapeDtypeStruct((1, E), _f32),
                   jax.ShapeDtypeStruct((NP, 32), _f32)),
        scratch_shapes=[pltpu.VMEM((1, HID), _f32)],
    )(population.reshape(1, POP), fc11_W, fc11_b.reshape(1, HID),
      fc22_W, fc22_b.reshape(1, E), x, conv1_W)


def _deg_body(dst_hbm, ew_hbm, out_ref, deg_sh, dstv, ewv, bounce, sem):
    c = lax.axis_index("c")
    s = lax.axis_index("s")
    w = s * NC + c

    def _z(i, carry):
        bounce[pl.ds(i * 16, 16)] = jnp.zeros((16,), _f32)
        return carry
    lax.fori_loop(0, ROWS_PER_TILE // 16, _z, 0)
    pltpu.sync_copy(bounce, deg_sh.at[pl.ds(s * ROWS_PER_TILE, ROWS_PER_TILE)])
    plsc.subcore_barrier()

    pltpu.sync_copy(dst_hbm.at[w], dstv)
    pltpu.sync_copy(ew_hbm.at[w], ewv)

    descs = []
    for j in range(CPW):
        descs.append(
            pltpu.async_copy(ewv.at[j], deg_sh.at[dstv.at[j]], sem, add=True))
    for d in descs:
        d.wait()
    plsc.subcore_barrier()

    pltpu.sync_copy(deg_sh.at[pl.ds(s * ROWS_PER_TILE, ROWS_PER_TILE)], bounce)
    pltpu.sync_copy(bounce, out_ref.at[c, pl.ds(s * ROWS_PER_TILE, ROWS_PER_TILE)])


def _deg_partials(dst2d, ew2d):
    mesh = plsc.VectorSubcoreMesh(core_axis_name="c", subcore_axis_name="s",
                                  num_cores=NC, num_subcores=NS)
    return pl.kernel(
        _deg_body,
        out_type=jax.ShapeDtypeStruct((NC, NP), _f32),
        mesh=mesh,
        compiler_params=pltpu.CompilerParams(use_tc_tiling_on_sc=False),
        scratch_types=[
            pltpu.VMEM_SHARED((NP,), _f32),
            pltpu.VMEM((CPW, CH), jnp.int32),
            pltpu.VMEM((CPW, CH), _f32),
            pltpu.VMEM((ROWS_PER_TILE,), _f32),
            pltpu.SemaphoreType.DMA,
        ],
    )(dst2d, ew2d)


def _conv_body(F, t_hbm, src_hbm, dst_hbm, ew_hbm, out_ref,
               acc_sh, t_sh, srcv, dstv, ewv, rows_a, rows_b, rows_c, rows_d,
               gsem, ssem):
    c = lax.axis_index("c")
    s = lax.axis_index("s")
    w = s * NC + c
    nblk = F // 16

    def _z(r, carry):
        for k in range(nblk):
            rows_a[r, pl.ds(k * 16, 16)] = jnp.zeros((16,), _f32)
        return carry
    lax.fori_loop(0, DCH, _z, 0)
    for k in range(ROWS_PER_TILE // DCH):
        sl = pl.ds(s * ROWS_PER_TILE + k * DCH, DCH)
        pltpu.sync_copy(rows_a.at[pl.ds(0, DCH), :], acc_sh.at[sl, :])
        pltpu.sync_copy(t_hbm.at[sl, :], rows_b.at[pl.ds(0, DCH), :])
        pltpu.sync_copy(rows_b.at[pl.ds(0, DCH), :], t_sh.at[sl, :])
    plsc.subcore_barrier()

    pltpu.sync_copy(src_hbm.at[w], srcv)
    pltpu.sync_copy(dst_hbm.at[w], dstv)
    pltpu.sync_copy(ew_hbm.at[w], ewv)

    bufs = (rows_a, rows_b, rows_c, rows_d)
    nbuf = len(bufs)

    def _scale(j, buf):
        def _grp(g, carry):
            ev = ewv[j, pl.ds(g * 16, 16)]
            for k in range(16):
                r = g * 16 + k
                e = ev[k]
                for b in range(nblk):
                    buf[r, pl.ds(b * 16, 16)] = buf[r, pl.ds(b * 16, 16)] * e
            return carry
        lax.fori_loop(0, CH // 16, _grp, 0)

    g = {}
    sdesc = {}
    for j in range(nbuf - 1):
        g[j] = pltpu.async_copy(t_sh.at[srcv.at[j]], bufs[j], gsem)
    for j in range(CPW):
        g[j].wait()
        _scale(j, bufs[j % nbuf])
        sdesc[j] = pltpu.async_copy(bufs[j % nbuf], acc_sh.at[dstv.at[j]],
                                    ssem, add=True)
        jn = j + nbuf - 1
        if jn < CPW:
            if jn - nbuf >= 0:
                sdesc[jn - nbuf].wait()
            g[jn] = pltpu.async_copy(t_sh.at[srcv.at[jn]],
                                     bufs[jn % nbuf], gsem)
    for j in range(CPW - nbuf, CPW):
        sdesc[j].wait()
    plsc.subcore_barrier()

    for k in range(ROWS_PER_TILE // DCH):
        sl = pl.ds(s * ROWS_PER_TILE + k * DCH, DCH)
        pltpu.sync_copy(acc_sh.at[sl, :], rows_a.at[pl.ds(0, DCH), :])
        pltpu.sync_copy(rows_a.at[pl.ds(0, DCH), :], out_ref.at[c, sl, :])


def _conv_partials(t, src2d, dst2d, ew2d, F):
    mesh = plsc.VectorSubcoreMesh(core_axis_name="c", subcore_axis_name="s",
                                  num_cores=NC, num_subcores=NS)
    return pl.kernel(
        functools.partial(_conv_body, F),
        out_type=jax.ShapeDtypeStruct((NC, NP, F), _f32),
        mesh=mesh,
        compiler_params=pltpu.CompilerParams(use_tc_tiling_on_sc=False),
        scratch_types=[
            pltpu.VMEM_SHARED((NP, F), _f32),
            pltpu.VMEM_SHARED((NP, F), _f32),
            pltpu.VMEM((CPW, CH), jnp.int32),
            pltpu.VMEM((CPW, CH), jnp.int32),
            pltpu.VMEM((CPW, CH), _f32),
            pltpu.VMEM((CH, F), _f32),
            pltpu.VMEM((CH, F), _f32),
            pltpu.VMEM((CH, F), _f32),
            pltpu.VMEM((CH, F), _f32),
            pltpu.SemaphoreType.DMA,
            pltpu.SemaphoreType.DMA,
        ],
    )(t, src2d, dst2d, ew2d)


def _dis_t1_body(degp_ref, xw1_ref, dis_ref, t1_ref):
    dp = degp_ref[...]
    deg = 1.0 + dp[0] + dp[1]
    dis = lax.rsqrt(deg).reshape(NP, 1)
    dis_ref[...] = dis
    t1_ref[...] = dis * xw1_ref[...]


def _dis_t1(degp, xw1):
    return pl.pallas_call(
        _dis_t1_body,
        out_shape=(jax.ShapeDtypeStruct((NP, 1), _f32),
                   jax.ShapeDtypeStruct((NP, 32), _f32)),
    )(degp, xw1)


def _mid_body(accp_ref, t1_ref, dis_ref, b1_ref, w2_ref, h_ref, t2_ref):
    ap = accp_ref[...]
    dis = dis_ref[...]
    out1 = dis * (ap[0] + ap[1] + t1_ref[...]) + b1_ref[...]
    h = jnp.maximum(out1, 0.0)
    h_ref[...] = h
    xw2 = lax.dot_general(h, w2_ref[...], (((1,), (1,)), ((), ())),
                          preferred_element_type=_f32)
    t2_ref[...] = dis * xw2


def _mid(accp1, t1, dis, conv1_b, conv2_W):
    return pl.pallas_call(
        _mid_body,
        out_shape=(jax.ShapeDtypeStruct((NP, 32), _f32),
                   jax.ShapeDtypeStruct((NP, 16), _f32)),
    )(accp1, t1, dis, conv1_b.reshape(1, 32), conv2_W)


def _head_body(accp_ref, t2_ref, dis_ref, b2_ref, h_ref, f1w_ref, f1b_ref,
               f2wh_ref, f2wc_ref, f2b_ref, x1o_ref, r_ref):
    ap = accp_ref[...]
    x2 = dis_ref[...] * (ap[0] + ap[1] + t2_ref[...]) + b2_ref[...]
    x1o = jnp.sum(x2 * f1w_ref[...], axis=1, keepdims=True) + f1b_ref[...]
    x1o_ref[...] = x1o
    cc = jax.nn.sigmoid(x1o)
    r_ref[...] = (jnp.sum(h_ref[...] * f2wh_ref[...], axis=1, keepdims=True)
                  + cc * f2wc_ref[...] + f2b_ref[...])


def _heads(accp2, t2, dis, conv2_b, h, fc1_W, fc1_b, fc2_W, fc2_b):
    return pl.pallas_call(
        _head_body,
        out_shape=(jax.ShapeDtypeStruct((NP, 1), _f32),
                   jax.ShapeDtypeStruct((NP, 1), _f32)),
    )(accp2, t2, dis, conv2_b.reshape(1, 16), h,
      fc1_W, fc1_b.reshape(1, 1), fc2_W[:, :32], fc2_W[:, 32:33],
      fc2_b.reshape(1, 1))


@jax.jit
def kernel(x, edge_index, population, fc11_W, fc11_b, fc22_W, fc22_b,
           conv1_W, conv1_b, conv2_W, conv2_b, fc1_W, fc1_b, fc2_W, fc2_b):
    ew2, xw1 = _edge_weights_xw1(population, fc11_W, fc11_b, fc22_W, fc22_b,
                                 x, conv1_W)
    ew = ew2[0]

    pad = EP - E
    zi = jnp.zeros((pad,), jnp.int32)
    src2d = jnp.concatenate([edge_index[0], zi]).reshape(NW, CPW, CH)
    dst2d = jnp.concatenate([edge_index[1], zi]).reshape(NW, CPW, CH)
    ew2d = jnp.concatenate([ew, jnp.zeros((pad,), _f32)]).reshape(NW, CPW, CH)

    degp = _deg_partials(dst2d, ew2d)
    dis, t1 = _dis_t1(degp, xw1)
    accp1 = _conv_partials(t1, src2d, dst2d, ew2d, 32)
    h, t2 = _mid(accp1, t1, dis, conv1_b, conv2_W)
    accp2 = _conv_partials(t2, src2d, dst2d, ew2d, 16)
    x1o, r = _heads(accp2, t2, dis, conv2_b, h, fc1_W, fc1_b, fc2_W, fc2_b)
    return (r[:N], x1o[:N])


# speedup vs baseline: 1.0123x; 1.0123x over previous
import functools

import jax
import jax.numpy as jnp
from jax import lax
from jax.experimental import pallas as pl
from jax.experimental.pallas import tpu as pltpu
from jax.experimental.pallas import tpu_sc as plsc

N = 10000
E = 160000
F_IN = 256
POP = 1024
HID = 512

NC = 2
NS = 16
NW = NC * NS
NP = 10240
CH = 512
EP = 163840
CPW = EP // NW // CH
ROWS_PER_TILE = NP // NS
DCH = 128

_f32 = jnp.float32


def _ew_body(pop_ref, f11w_ref, f11b_ref, f22w_ref, f22b_ref, out_ref, h2_ref):
    i = pl.program_id(0)

    @pl.when(i == 0)
    def _():
        h2 = lax.dot_general(pop_ref[...], f11w_ref[...], (((1,), (1,)), ((), ())),
                             preferred_element_type=_f32)
        h2_ref[...] = h2 + f11b_ref[...]

    y = lax.dot_general(h2_ref[...], f22w_ref[...], (((1,), (1,)), ((), ())),
                        preferred_element_type=_f32)
    out_ref[...] = jax.nn.sigmoid(y + f22b_ref[...])


def _edge_weights(population, fc11_W, fc11_b, fc22_W, fc22_b):
    RB = 6400
    grid = E // RB
    return pl.pallas_call(
        _ew_body,
        grid=(grid,),
        in_specs=[
            pl.BlockSpec((1, POP), lambda i: (0, 0)),
            pl.BlockSpec((HID, POP), lambda i: (0, 0)),
            pl.BlockSpec((1, HID), lambda i: (0, 0)),
            pl.BlockSpec((RB, HID), lambda i: (i, 0)),
            pl.BlockSpec((1, RB), lambda i: (0, i)),
        ],
        out_specs=pl.BlockSpec((1, RB), lambda i: (0, i)),
        out_shape=jax.ShapeDtypeStruct((1, E), _f32),
        scratch_shapes=[pltpu.VMEM((1, HID), _f32)],
    )(population.reshape(1, POP), fc11_W, fc11_b.reshape(1, HID),
      fc22_W, fc22_b.reshape(1, E))


def _xw1_body(x_ref, w_ref, out_ref):
    xw = lax.dot_general(x_ref[...], w_ref[...], (((1,), (1,)), ((), ())),
                         preferred_element_type=_f32)
    out_ref[pl.ds(0, N), :] = xw
    out_ref[pl.ds(N, NP - N), :] = jnp.zeros((NP - N, 32), _f32)


def _xw1(x, conv1_W):
    return pl.pallas_call(
        _xw1_body,
        out_shape=jax.ShapeDtypeStruct((NP, 32), _f32),
    )(x, conv1_W)


def _deg_body(dst_hbm, ew_hbm, out_ref, deg_sh, dstv, ewv, bounce, sem):
    c = lax.axis_index("c")
    s = lax.axis_index("s")
    w = s * NC + c

    def _z(i, carry):
        bounce[pl.ds(i * 16, 16)] = jnp.zeros((16,), _f32)
        return carry
    lax.fori_loop(0, ROWS_PER_TILE // 16, _z, 0)
    pltpu.sync_copy(bounce, deg_sh.at[pl.ds(s * ROWS_PER_TILE, ROWS_PER_TILE)])
    plsc.subcore_barrier()

    pltpu.sync_copy(dst_hbm.at[w], dstv)
    pltpu.sync_copy(ew_hbm.at[w], ewv)

    descs = []
    for j in range(CPW):
        descs.append(
            pltpu.async_copy(ewv.at[j], deg_sh.at[dstv.at[j]], sem, add=True))
    for d in descs:
        d.wait()
    plsc.subcore_barrier()

    pltpu.sync_copy(deg_sh.at[pl.ds(s * ROWS_PER_TILE, ROWS_PER_TILE)], bounce)
    pltpu.sync_copy(bounce, out_ref.at[c, pl.ds(s * ROWS_PER_TILE, ROWS_PER_TILE)])


def _deg_partials(dst2d, ew2d):
    mesh = plsc.VectorSubcoreMesh(core_axis_name="c", subcore_axis_name="s",
                                  num_cores=NC, num_subcores=NS)
    return pl.kernel(
        _deg_body,
        out_type=jax.ShapeDtypeStruct((NC, NP), _f32),
        mesh=mesh,
        compiler_params=pltpu.CompilerParams(use_tc_tiling_on_sc=False),
        scratch_types=[
            pltpu.VMEM_SHARED((NP,), _f32),
            pltpu.VMEM((CPW, CH), jnp.int32),
            pltpu.VMEM((CPW, CH), _f32),
            pltpu.VMEM((ROWS_PER_TILE,), _f32),
            pltpu.SemaphoreType.DMA,
        ],
    )(dst2d, ew2d)


def _conv_body(F, t_hbm, src_hbm, dst_hbm, ew_hbm, out_ref,
               acc_sh, t_sh, srcv, dstv, ewv, rows_a, rows_b, rows_c, rows_d,
               gsem, ssem):
    c = lax.axis_index("c")
    s = lax.axis_index("s")
    w = s * NC + c
    nblk = F // 16

    def _z(r, carry):
        for k in range(nblk):
            rows_a[r, pl.ds(k * 16, 16)] = jnp.zeros((16,), _f32)
        return carry
    lax.fori_loop(0, DCH, _z, 0)
    for k in range(ROWS_PER_TILE // DCH):
        sl = pl.ds(s * ROWS_PER_TILE + k * DCH, DCH)
        pltpu.sync_copy(rows_a.at[pl.ds(0, DCH), :], acc_sh.at[sl, :])
        pltpu.sync_copy(t_hbm.at[sl, :], rows_b.at[pl.ds(0, DCH), :])
        pltpu.sync_copy(rows_b.at[pl.ds(0, DCH), :], t_sh.at[sl, :])
    plsc.subcore_barrier()

    pltpu.sync_copy(src_hbm.at[w], srcv)
    pltpu.sync_copy(dst_hbm.at[w], dstv)
    pltpu.sync_copy(ew_hbm.at[w], ewv)

    bufs = (rows_a, rows_b, rows_c, rows_d)
    nbuf = len(bufs)

    def _scale(j, buf):
        def _grp(g, carry):
            ev = ewv[j, pl.ds(g * 16, 16)]
            for k in range(16):
                r = g * 16 + k
                e = ev[k]
                for b in range(nblk):
                    buf[r, pl.ds(b * 16, 16)] = buf[r, pl.ds(b * 16, 16)] * e
            return carry
        lax.fori_loop(0, CH // 16, _grp, 0)

    g = {}
    sdesc = {}
    for j in range(nbuf - 1):
        g[j] = pltpu.async_copy(t_sh.at[srcv.at[j]], bufs[j], gsem)
    for j in range(CPW):
        g[j].wait()
        _scale(j, bufs[j % nbuf])
        sdesc[j] = pltpu.async_copy(bufs[j % nbuf], acc_sh.at[dstv.at[j]],
                                    ssem, add=True)
        jn = j + nbuf - 1
        if jn < CPW:
            if jn - nbuf >= 0:
                sdesc[jn - nbuf].wait()
            g[jn] = pltpu.async_copy(t_sh.at[srcv.at[jn]],
                                     bufs[jn % nbuf], gsem)
    for j in range(CPW - nbuf, CPW):
        sdesc[j].wait()
    plsc.subcore_barrier()

    for k in range(ROWS_PER_TILE // DCH):
        sl = pl.ds(s * ROWS_PER_TILE + k * DCH, DCH)
        pltpu.sync_copy(acc_sh.at[sl, :], rows_a.at[pl.ds(0, DCH), :])
        pltpu.sync_copy(rows_a.at[pl.ds(0, DCH), :], out_ref.at[c, sl, :])


def _conv_partials(t, src2d, dst2d, ew2d, F):
    mesh = plsc.VectorSubcoreMesh(core_axis_name="c", subcore_axis_name="s",
                                  num_cores=NC, num_subcores=NS)
    return pl.kernel(
        functools.partial(_conv_body, F),
        out_type=jax.ShapeDtypeStruct((NC, NP, F), _f32),
        mesh=mesh,
        compiler_params=pltpu.CompilerParams(use_tc_tiling_on_sc=False),
        scratch_types=[
            pltpu.VMEM_SHARED((NP, F), _f32),
            pltpu.VMEM_SHARED((NP, F), _f32),
            pltpu.VMEM((CPW, CH), jnp.int32),
            pltpu.VMEM((CPW, CH), jnp.int32),
            pltpu.VMEM((CPW, CH), _f32),
            pltpu.VMEM((CH, F), _f32),
            pltpu.VMEM((CH, F), _f32),
            pltpu.VMEM((CH, F), _f32),
            pltpu.VMEM((CH, F), _f32),
            pltpu.SemaphoreType.DMA,
            pltpu.SemaphoreType.DMA,
        ],
    )(t, src2d, dst2d, ew2d)


def _dis_t1_body(degp_ref, xw1_ref, dis_ref, t1_ref):
    dp = degp_ref[...]
    deg = 1.0 + dp[0] + dp[1]
    dis = lax.rsqrt(deg).reshape(NP, 1)
    dis_ref[...] = dis
    t1_ref[...] = dis * xw1_ref[...]


def _dis_t1(degp, xw1):
    return pl.pallas_call(
        _dis_t1_body,
        out_shape=(jax.ShapeDtypeStruct((NP, 1), _f32),
                   jax.ShapeDtypeStruct((NP, 32), _f32)),
    )(degp, xw1)


def _mid_body(accp_ref, t1_ref, dis_ref, b1_ref, w2_ref, h_ref, t2_ref):
    ap = accp_ref[...]
    dis = dis_ref[...]
    out1 = dis * (ap[0] + ap[1] + t1_ref[...]) + b1_ref[...]
    h = jnp.maximum(out1, 0.0)
    h_ref[...] = h
    xw2 = lax.dot_general(h, w2_ref[...], (((1,), (1,)), ((), ())),
                          preferred_element_type=_f32)
    t2_ref[...] = dis * xw2


def _mid(accp1, t1, dis, conv1_b, conv2_W):
    return pl.pallas_call(
        _mid_body,
        out_shape=(jax.ShapeDtypeStruct((NP, 32), _f32),
                   jax.ShapeDtypeStruct((NP, 16), _f32)),
    )(accp1, t1, dis, conv1_b.reshape(1, 32), conv2_W)


def _head_body(accp_ref, t2_ref, dis_ref, b2_ref, h_ref, f1w_ref, f1b_ref,
               f2wh_ref, f2wc_ref, f2b_ref, x1o_ref, r_ref):
    ap = accp_ref[...]
    x2 = dis_ref[...] * (ap[0] + ap[1] + t2_ref[...]) + b2_ref[...]
    x1o = jnp.sum(x2 * f1w_ref[...], axis=1, keepdims=True) + f1b_ref[...]
    x1o_ref[...] = x1o
    cc = jax.nn.sigmoid(x1o)
    r_ref[...] = (jnp.sum(h_ref[...] * f2wh_ref[...], axis=1, keepdims=True)
                  + cc * f2wc_ref[...] + f2b_ref[...])


def _heads(accp2, t2, dis, conv2_b, h, fc1_W, fc1_b, fc2_W, fc2_b):
    return pl.pallas_call(
        _head_body,
        out_shape=(jax.ShapeDtypeStruct((NP, 1), _f32),
                   jax.ShapeDtypeStruct((NP, 1), _f32)),
    )(accp2, t2, dis, conv2_b.reshape(1, 16), h,
      fc1_W, fc1_b.reshape(1, 1), fc2_W[:, :32], fc2_W[:, 32:33],
      fc2_b.reshape(1, 1))


@jax.jit
def kernel(x, edge_index, population, fc11_W, fc11_b, fc22_W, fc22_b,
           conv1_W, conv1_b, conv2_W, conv2_b, fc1_W, fc1_b, fc2_W, fc2_b):
    ew = _edge_weights(population, fc11_W, fc11_b, fc22_W, fc22_b)[0]

    pad = EP - E
    zi = jnp.zeros((pad,), jnp.int32)
    src2d = jnp.concatenate([edge_index[0], zi]).reshape(NW, CPW, CH)
    dst2d = jnp.concatenate([edge_index[1], zi]).reshape(NW, CPW, CH)
    ew2d = jnp.concatenate([ew, jnp.zeros((pad,), _f32)]).reshape(NW, CPW, CH)

    xw1 = _xw1(x, conv1_W)
    degp = _deg_partials(dst2d, ew2d)
    dis, t1 = _dis_t1(degp, xw1)
    accp1 = _conv_partials(t1, src2d, dst2d, ew2d, 32)
    h, t2 = _mid(accp1, t1, dis, conv1_b, conv2_W)
    accp2 = _conv_partials(t2, src2d, dst2d, ew2d, 16)
    x1o, r = _heads(accp2, t2, dis, conv2_b, h, fc1_W, fc1_b, fc2_W, fc2_b)
    return (r[:N], x1o[:N])
